# baseline (speedup 1.0000x reference)
import functools

import jax
import jax.numpy as jnp
from jax import lax
from jax.experimental import pallas as pl
from jax.experimental.pallas import tpu as pltpu
from jax.experimental.pallas import tpu_sc as plsc

_ALPHA = 4


@functools.lru_cache(maxsize=None)
def _make_sc_gather(C, T, H, W):
    S = T // _ALPHA
    info = plsc.get_sparse_core_info()
    NW = info.num_cores * info.num_subcores
    NFR = C * S
    CHN = 1
    while ((NFR * CHN) % NW != 0 or (H // CHN) * W > 49152
           or H % CHN != 0 or (H // CHN) % 8 != 0):
        CHN += 1
    ROWS = H // CHN
    PPW = (NFR * CHN) // NW

    mesh = plsc.ScalarSubcoreMesh(axis_name="c", num_cores=info.num_cores)
    FPC = NFR // info.num_cores

    @functools.partial(
        pl.kernel,
        mesh=mesh,
        out_type=jax.ShapeDtypeStruct((C, S, H, W), jnp.float32),
        scratch_types=(
            [pltpu.VMEM_SHARED((4, H, W), jnp.float32)]
            + [pltpu.SemaphoreType.DMA] * 8
        ),
    )
    def gather(frames_hbm, out_hbm, buf, *sems):
        NB = 4
        cid = lax.axis_index("c")
        isems = sems[:NB]
        osems = sems[NB:]

        def coords(p):
            pid = cid * FPC + p
            c = pid // S
            j = pid % S
            t = (j * (T - 1)) // (S - 1)
            return c, t, j

        def cp_in(p):
            c, t, _ = coords(p)
            return pltpu.make_async_copy(
                frames_hbm.at[c, t], buf.at[p % NB], isems[p % NB])

        def cp_out(p):
            c, _, j = coords(p)
            return pltpu.make_async_copy(
                buf.at[p % NB], out_hbm.at[c, j], osems[p % NB])

        out_cp = [None] * FPC
        in_cp = [None] * FPC
        for p in range(FPC):
            if p >= NB:
                out_cp[p - NB].wait()
            in_cp[p] = cp_in(p)
            in_cp[p].start()
            if p >= 1:
                in_cp[p - 1].wait()
                out_cp[p - 1] = cp_out(p - 1)
                out_cp[p - 1].start()
        in_cp[FPC - 1].wait()
        out_cp[FPC - 1] = cp_out(FPC - 1)
        out_cp[FPC - 1].start()
        for p in range(max(FPC - NB, 0), FPC):
            out_cp[p].wait()

    return gather


@functools.lru_cache(maxsize=None)
def _make_tc_copy(C, T, H, W, BT=16):
    def body(i_ref, o_ref):
        o_ref[...] = i_ref[...]

    return pl.pallas_call(
        body,
        grid=(C, T // BT),
        in_specs=[pl.BlockSpec((1, BT, H, W), lambda c, t: (c, t, 0, 0))],
        out_specs=pl.BlockSpec((1, BT, H, W), lambda c, t: (c, t, 0, 0)),
        out_shape=jax.ShapeDtypeStruct((C, T, H, W), jnp.float32),
    )


def kernel(frames):
    C, T, H, W = frames.shape
    fast = _make_tc_copy(C, T, H, W)(frames)
    slow = _make_sc_gather(C, T, H, W)(frames)
    return (slow, fast)


# speedup vs baseline: 1.0028x; 1.0028x over previous
import functools

import jax
import jax.numpy as jnp
from jax import lax
from jax.experimental import pallas as pl
from jax.experimental.pallas import tpu as pltpu
from jax.experimental.pallas import tpu_sc as plsc

_ALPHA = 4


@functools.lru_cache(maxsize=None)
def _make_sc_gather(C, T, H, W):
    S = T // _ALPHA
    info = plsc.get_sparse_core_info()
    NFR = C * S
    mesh = plsc.ScalarSubcoreMesh(axis_name="c", num_cores=info.num_cores)
    FPC = NFR // info.num_cores
    NB = 4

    @functools.partial(
        pl.kernel,
        mesh=mesh,
        out_type=jax.ShapeDtypeStruct((C, S, H, W), jnp.float32),
        scratch_types=(
            [pltpu.VMEM_SHARED((NB, H, W), jnp.float32)]
            + [pltpu.SemaphoreType.DMA] * (2 * NB)
        ),
    )
    def gather(frames_hbm, out_hbm, buf, *sems):
        cid = lax.axis_index("c")
        isems = sems[:NB]
        osems = sems[NB:]

        def coords(p):
            pid = cid * FPC + p
            c = pid // S
            j = pid % S
            t = (j * (T - 1)) // (S - 1)
            return c, t, j

        def cp_in(p):
            c, t, _ = coords(p)
            return pltpu.make_async_copy(
                frames_hbm.at[c, t], buf.at[p % NB], isems[p % NB])

        def cp_out(p):
            c, _, j = coords(p)
            return pltpu.make_async_copy(
                buf.at[p % NB], out_hbm.at[c, j], osems[p % NB])

        out_cp = [None] * FPC
        in_cp = [None] * FPC
        for p in range(FPC):
            if p >= NB:
                out_cp[p - NB].wait()
            in_cp[p] = cp_in(p)
            in_cp[p].start()
            if p >= 1:
                in_cp[p - 1].wait()
                out_cp[p - 1] = cp_out(p - 1)
                out_cp[p - 1].start()
        in_cp[FPC - 1].wait()
        out_cp[FPC - 1] = cp_out(FPC - 1)
        out_cp[FPC - 1].start()
        for p in range(max(FPC - NB, 0), FPC):
            out_cp[p].wait()

    return gather


@functools.lru_cache(maxsize=None)
def _make_tc_copy(C, T, H, W, BT=16):
    def body(i_ref, o_ref):
        o_ref[...] = i_ref[...]

    return pl.pallas_call(
        body,
        grid=(C, T // BT),
        in_specs=[pl.BlockSpec((1, BT, H, W), lambda c, t: (c, t, 0, 0))],
        out_specs=pl.BlockSpec((1, BT, H, W), lambda c, t: (c, t, 0, 0)),
        out_shape=jax.ShapeDtypeStruct((C, T, H, W), jnp.float32),
    )


def kernel(frames):
    C, T, H, W = frames.shape
    fast = _make_tc_copy(C, T, H, W)(frames)
    slow = _make_sc_gather(C, T, H, W)(frames)
    return (slow, fast)
